# baseline (speedup 1.0000x reference)
import functools

import jax
import jax.numpy as jnp
from jax import lax
from jax.experimental import pallas as pl
from jax.experimental.pallas import tpu as pltpu
from jax.experimental.pallas import tpu_sc as plsc

MODEL_DIM = 64
NUM_CORES = 2
NUM_SUBCORES = 16
NUM_WORKERS = NUM_CORES * NUM_SUBCORES
CHUNK = 128
SCALE = 8.0


def _sc_embedding_lookup(table, idx3):
    n_chunks = idx3.shape[1]
    rows_per_w = n_chunks * CHUNK
    total = NUM_WORKERS * rows_per_w

    mesh = plsc.VectorSubcoreMesh(core_axis_name="c", subcore_axis_name="s")

    @functools.partial(
        pl.kernel,
        mesh=mesh,
        out_type=jax.ShapeDtypeStruct((total, MODEL_DIM), jnp.float32),
        scratch_types=[
            pltpu.VMEM((n_chunks, CHUNK), jnp.int32),
            pltpu.VMEM((CHUNK, MODEL_DIM), jnp.float32),
            pltpu.SemaphoreType.DMA,
        ],
        compiler_params=pltpu.CompilerParams(use_tc_tiling_on_sc=False),
    )
    def k(table_hbm, idx_hbm, out_hbm, idx_v, rows_v, sem):
        cid = lax.axis_index("c")
        sid = lax.axis_index("s")
        wid = sid * NUM_CORES + cid
        base = wid * rows_per_w
        pltpu.sync_copy(idx_hbm.at[wid], idx_v)

        def chunk_body(c, carry):
            pltpu.async_copy(table_hbm.at[idx_v.at[c]], rows_v, sem).wait()

            def row_body(r, rc):
                for cc in range(MODEL_DIM // 16):
                    sl = pl.ds(cc * 16, 16)
                    rows_v[r, sl] = rows_v[r, sl] * SCALE
                return rc

            lax.fori_loop(0, CHUNK, row_body, 0)
            pltpu.sync_copy(rows_v, out_hbm.at[pl.ds(base + c * CHUNK, CHUNK)])
            return carry

        lax.fori_loop(0, n_chunks, chunk_body, 0)

    return k(table, idx3)


def kernel(token_indices, embeddings):
    b, s = token_indices.shape
    total = b * s
    rows_per_w = total // NUM_WORKERS
    n_chunks = rows_per_w // CHUNK
    idx3 = token_indices.reshape(NUM_WORKERS, n_chunks, CHUNK).astype(jnp.int32)
    out = _sc_embedding_lookup(embeddings, idx3)
    return out.reshape(b, s, MODEL_DIM)
